# baseline (speedup 1.0000x reference)
import jax
import jax.numpy as jnp
from jax import lax
from jax.experimental import pallas as pl
from jax.experimental.pallas import tpu as pltpu
from jax.experimental.pallas import tpu_sc as plsc

BATCH = 4096
DIM = 128
LANES = 16
NUM_WORKERS = 32
B_PER_W = BATCH // NUM_WORKERS
HALF = B_PER_W // 2
CHUNKS = DIM // LANES


def _body(
    session_idx_hbm,
    subject_idx_hbm,
    session_table_hbm,
    subject_table_hbm,
    session_flag_hbm,
    subject_flag_hbm,
    out_hbm,
    idx_s_v,
    idx_u_v,
    rows_s_v,
    rows_u_v,
    flag_s_v,
    flag_u_v,
    sem_is,
    sem_iu,
    sem_flag,
    sem_s0,
    sem_s1,
    sem_u0,
    sem_u1,
    sem_out,
):
    wid = lax.axis_index("s") * 2 + lax.axis_index("c")
    base = wid * B_PER_W

    cp_is = pltpu.async_copy(session_idx_hbm.at[pl.ds(base, B_PER_W)], idx_s_v, sem_is)
    cp_iu = pltpu.async_copy(subject_idx_hbm.at[pl.ds(base, B_PER_W)], idx_u_v, sem_iu)
    cp_fs = pltpu.async_copy(session_flag_hbm, flag_s_v, sem_flag)
    cp_fu = pltpu.async_copy(subject_flag_hbm, flag_u_v, sem_flag)

    cp_is.wait()
    g_s0 = pltpu.async_copy(
        session_table_hbm.at[idx_s_v.at[pl.ds(0, HALF)]], rows_s_v.at[pl.ds(0, HALF)], sem_s0
    )
    g_s1 = pltpu.async_copy(
        session_table_hbm.at[idx_s_v.at[pl.ds(HALF, HALF)]], rows_s_v.at[pl.ds(HALF, HALF)], sem_s1
    )
    cp_iu.wait()
    g_u0 = pltpu.async_copy(
        subject_table_hbm.at[idx_u_v.at[pl.ds(0, HALF)]], rows_u_v.at[pl.ds(0, HALF)], sem_u0
    )
    g_u1 = pltpu.async_copy(
        subject_table_hbm.at[idx_u_v.at[pl.ds(HALF, HALF)]], rows_u_v.at[pl.ds(HALF, HALF)], sem_u1
    )

    cp_fs.wait()
    cp_fu.wait()
    fl_s = [flag_s_v[pl.ds(c * LANES, LANES)] for c in range(CHUNKS)]
    fl_u = [flag_u_v[pl.ds(c * LANES, LANES)] for c in range(CHUNKS)]

    def add_flags(rows_v, fl, lo):
        @plsc.parallel_loop(lo, lo + HALF, unroll=4)
        def _(r):
            for c in range(CHUNKS):
                sl = pl.ds(c * LANES, LANES)
                rows_v[r, sl] = rows_v[r, sl] + fl[c]

    g_s0.wait()
    add_flags(rows_s_v, fl_s, 0)
    p1 = pltpu.async_copy(
        rows_s_v.at[pl.ds(0, HALF)], out_hbm.at[pl.ds(base, HALF), 0], sem_out
    )
    g_s1.wait()
    add_flags(rows_s_v, fl_s, HALF)
    p2 = pltpu.async_copy(
        rows_s_v.at[pl.ds(HALF, HALF)], out_hbm.at[pl.ds(base + HALF, HALF), 0], sem_out
    )
    g_u0.wait()
    add_flags(rows_u_v, fl_u, 0)
    p3 = pltpu.async_copy(
        rows_u_v.at[pl.ds(0, HALF)], out_hbm.at[pl.ds(base, HALF), 1], sem_out
    )
    g_u1.wait()
    add_flags(rows_u_v, fl_u, HALF)
    p4 = pltpu.async_copy(
        rows_u_v.at[pl.ds(HALF, HALF)], out_hbm.at[pl.ds(base + HALF, HALF), 1], sem_out
    )

    p1.wait()
    p2.wait()
    p3.wait()
    p4.wait()


@jax.jit
def kernel(session_idx, subject_idx, session_table, subject_table, session_flag, subject_flag):
    mesh = plsc.VectorSubcoreMesh(core_axis_name="c", subcore_axis_name="s")
    run = pl.kernel(
        _body,
        out_type=jax.ShapeDtypeStruct((BATCH, 2, DIM), jnp.float32),
        mesh=mesh,
        scratch_types=[
            pltpu.VMEM((B_PER_W,), jnp.int32),
            pltpu.VMEM((B_PER_W,), jnp.int32),
            pltpu.VMEM((B_PER_W, DIM), jnp.float32),
            pltpu.VMEM((B_PER_W, DIM), jnp.float32),
            pltpu.VMEM((DIM,), jnp.float32),
            pltpu.VMEM((DIM,), jnp.float32),
            pltpu.SemaphoreType.DMA,
            pltpu.SemaphoreType.DMA,
            pltpu.SemaphoreType.DMA,
            pltpu.SemaphoreType.DMA,
            pltpu.SemaphoreType.DMA,
            pltpu.SemaphoreType.DMA,
            pltpu.SemaphoreType.DMA,
            pltpu.SemaphoreType.DMA,
        ],
    )
    return run(
        session_idx.astype(jnp.int32),
        subject_idx.astype(jnp.int32),
        session_table,
        subject_table,
        session_flag,
        subject_flag,
    )


# speedup vs baseline: 1.0173x; 1.0173x over previous
import jax
import jax.numpy as jnp
from jax import lax
from jax.experimental import pallas as pl
from jax.experimental.pallas import tpu as pltpu
from jax.experimental.pallas import tpu_sc as plsc

BATCH = 4096
DIM = 128
LANES = 16
NUM_WORKERS = 32
B_PER_W = BATCH // NUM_WORKERS
HALF = B_PER_W // 2
CHUNKS = DIM // LANES


def _body(
    session_idx_hbm,
    subject_idx_hbm,
    session_table_hbm,
    subject_table_hbm,
    session_flag_hbm,
    subject_flag_hbm,
    out_hbm,
    idx_s_v,
    idx_u_v,
    rows_s_v,
    rows_u_v,
    flag_s_v,
    flag_u_v,
    sem_is,
    sem_iu,
    sem_flag,
    sem_s0,
    sem_s1,
    sem_u0,
    sem_u1,
    sem_out,
):
    wid = lax.axis_index("s") * 2 + lax.axis_index("c")
    base = wid * B_PER_W

    cp_is = pltpu.async_copy(session_idx_hbm.at[pl.ds(base, B_PER_W)], idx_s_v, sem_is)
    cp_iu = pltpu.async_copy(subject_idx_hbm.at[pl.ds(base, B_PER_W)], idx_u_v, sem_iu)
    cp_fs = pltpu.async_copy(session_flag_hbm, flag_s_v, sem_flag)
    cp_fu = pltpu.async_copy(subject_flag_hbm, flag_u_v, sem_flag)

    cp_is.wait()
    g_s0 = pltpu.async_copy(
        session_table_hbm.at[idx_s_v.at[pl.ds(0, HALF)]], rows_s_v.at[pl.ds(0, HALF)], sem_s0
    )
    g_s1 = pltpu.async_copy(
        session_table_hbm.at[idx_s_v.at[pl.ds(HALF, HALF)]], rows_s_v.at[pl.ds(HALF, HALF)], sem_s1
    )
    cp_iu.wait()
    g_u0 = pltpu.async_copy(
        subject_table_hbm.at[idx_u_v.at[pl.ds(0, HALF)]], rows_u_v.at[pl.ds(0, HALF)], sem_u0
    )
    g_u1 = pltpu.async_copy(
        subject_table_hbm.at[idx_u_v.at[pl.ds(HALF, HALF)]], rows_u_v.at[pl.ds(HALF, HALF)], sem_u1
    )

    cp_fs.wait()
    cp_fu.wait()
    fl_s = [flag_s_v[pl.ds(c * LANES, LANES)] for c in range(CHUNKS)]
    fl_u = [flag_u_v[pl.ds(c * LANES, LANES)] for c in range(CHUNKS)]

    def add_flags(rows_v, fl, lo):
        @plsc.parallel_loop(lo, lo + HALF, unroll=2)
        def _(r):
            for c in range(CHUNKS):
                sl = pl.ds(c * LANES, LANES)
                rows_v[r, sl] = rows_v[r, sl] + fl[c]

    g_s0.wait()
    add_flags(rows_s_v, fl_s, 0)
    p1 = pltpu.async_copy(
        rows_s_v.at[pl.ds(0, HALF)], out_hbm.at[pl.ds(base, HALF), 0], sem_out
    )
    g_s1.wait()
    add_flags(rows_s_v, fl_s, HALF)
    p2 = pltpu.async_copy(
        rows_s_v.at[pl.ds(HALF, HALF)], out_hbm.at[pl.ds(base + HALF, HALF), 0], sem_out
    )
    g_u0.wait()
    add_flags(rows_u_v, fl_u, 0)
    p3 = pltpu.async_copy(
        rows_u_v.at[pl.ds(0, HALF)], out_hbm.at[pl.ds(base, HALF), 1], sem_out
    )
    g_u1.wait()
    add_flags(rows_u_v, fl_u, HALF)
    p4 = pltpu.async_copy(
        rows_u_v.at[pl.ds(HALF, HALF)], out_hbm.at[pl.ds(base + HALF, HALF), 1], sem_out
    )

    p1.wait()
    p2.wait()
    p3.wait()
    p4.wait()


@jax.jit
def kernel(session_idx, subject_idx, session_table, subject_table, session_flag, subject_flag):
    mesh = plsc.VectorSubcoreMesh(core_axis_name="c", subcore_axis_name="s")
    run = pl.kernel(
        _body,
        out_type=jax.ShapeDtypeStruct((BATCH, 2, DIM), jnp.float32),
        mesh=mesh,
        scratch_types=[
            pltpu.VMEM((B_PER_W,), jnp.int32),
            pltpu.VMEM((B_PER_W,), jnp.int32),
            pltpu.VMEM((B_PER_W, DIM), jnp.float32),
            pltpu.VMEM((B_PER_W, DIM), jnp.float32),
            pltpu.VMEM((DIM,), jnp.float32),
            pltpu.VMEM((DIM,), jnp.float32),
            pltpu.SemaphoreType.DMA,
            pltpu.SemaphoreType.DMA,
            pltpu.SemaphoreType.DMA,
            pltpu.SemaphoreType.DMA,
            pltpu.SemaphoreType.DMA,
            pltpu.SemaphoreType.DMA,
            pltpu.SemaphoreType.DMA,
            pltpu.SemaphoreType.DMA,
        ],
    )
    return run(
        session_idx.astype(jnp.int32),
        subject_idx.astype(jnp.int32),
        session_table,
        subject_table,
        session_flag,
        subject_flag,
    )


# speedup vs baseline: 1.0368x; 1.0192x over previous
import jax
import jax.numpy as jnp
from jax import lax
from jax.experimental import pallas as pl
from jax.experimental.pallas import tpu as pltpu
from jax.experimental.pallas import tpu_sc as plsc

BATCH = 4096
DIM = 128
LANES = 16
NUM_WORKERS = 32
B_PER_W = BATCH // NUM_WORKERS
HALF = B_PER_W // 2
CHUNKS = DIM // LANES


def _body(
    session_idx_hbm,
    subject_idx_hbm,
    session_table_hbm,
    subject_table_hbm,
    session_flag_hbm,
    subject_flag_hbm,
    out_hbm,
    idx_s_v,
    idx_u_v,
    rows_s_v,
    rows_u_v,
    flag_s_v,
    flag_u_v,
    sem_is,
    sem_iu,
    sem_flag,
    sem_s0,
    sem_s1,
    sem_u0,
    sem_u1,
    sem_out,
):
    wid = lax.axis_index("s") * 2 + lax.axis_index("c")
    base = wid * B_PER_W

    cp_is = pltpu.async_copy(session_idx_hbm.at[pl.ds(base, B_PER_W)], idx_s_v, sem_is)
    cp_iu = pltpu.async_copy(subject_idx_hbm.at[pl.ds(base, B_PER_W)], idx_u_v, sem_iu)
    cp_fs = pltpu.async_copy(session_flag_hbm, flag_s_v, sem_flag)
    cp_fu = pltpu.async_copy(subject_flag_hbm, flag_u_v, sem_flag)

    cp_is.wait()
    g_s0 = pltpu.async_copy(
        session_table_hbm.at[idx_s_v.at[pl.ds(0, HALF)]], rows_s_v.at[pl.ds(0, HALF)], sem_s0
    )
    g_s1 = pltpu.async_copy(
        session_table_hbm.at[idx_s_v.at[pl.ds(HALF, HALF)]], rows_s_v.at[pl.ds(HALF, HALF)], sem_s1
    )
    cp_iu.wait()
    g_u0 = pltpu.async_copy(
        subject_table_hbm.at[idx_u_v.at[pl.ds(0, HALF)]], rows_u_v.at[pl.ds(0, HALF)], sem_u0
    )
    g_u1 = pltpu.async_copy(
        subject_table_hbm.at[idx_u_v.at[pl.ds(HALF, HALF)]], rows_u_v.at[pl.ds(HALF, HALF)], sem_u1
    )

    cp_fs.wait()
    cp_fu.wait()
    fl_s = [flag_s_v[pl.ds(c * LANES, LANES)] for c in range(CHUNKS)]
    fl_u = [flag_u_v[pl.ds(c * LANES, LANES)] for c in range(CHUNKS)]

    def add_flags(rows_v, fl, lo):
        @plsc.parallel_loop(lo, lo + HALF, unroll=2)
        def _(r):
            for c in range(CHUNKS):
                sl = pl.ds(c * LANES, LANES)
                rows_v[r, sl] = rows_v[r, sl] + fl[c]

    g_s0.wait()
    add_flags(rows_s_v, fl_s, 0)
    g_s1.wait()
    add_flags(rows_s_v, fl_s, HALF)
    put_s = pltpu.async_copy(rows_s_v, out_hbm.at[pl.ds(base, B_PER_W), 0], sem_out)

    g_u0.wait()
    add_flags(rows_u_v, fl_u, 0)
    g_u1.wait()
    add_flags(rows_u_v, fl_u, HALF)
    put_u = pltpu.async_copy(rows_u_v, out_hbm.at[pl.ds(base, B_PER_W), 1], sem_out)

    put_s.wait()
    put_u.wait()


@jax.jit
def kernel(session_idx, subject_idx, session_table, subject_table, session_flag, subject_flag):
    mesh = plsc.VectorSubcoreMesh(core_axis_name="c", subcore_axis_name="s")
    run = pl.kernel(
        _body,
        out_type=jax.ShapeDtypeStruct((BATCH, 2, DIM), jnp.float32),
        mesh=mesh,
        scratch_types=[
            pltpu.VMEM((B_PER_W,), jnp.int32),
            pltpu.VMEM((B_PER_W,), jnp.int32),
            pltpu.VMEM((B_PER_W, DIM), jnp.float32),
            pltpu.VMEM((B_PER_W, DIM), jnp.float32),
            pltpu.VMEM((DIM,), jnp.float32),
            pltpu.VMEM((DIM,), jnp.float32),
            pltpu.SemaphoreType.DMA,
            pltpu.SemaphoreType.DMA,
            pltpu.SemaphoreType.DMA,
            pltpu.SemaphoreType.DMA,
            pltpu.SemaphoreType.DMA,
            pltpu.SemaphoreType.DMA,
            pltpu.SemaphoreType.DMA,
            pltpu.SemaphoreType.DMA,
        ],
    )
    return run(
        session_idx.astype(jnp.int32),
        subject_idx.astype(jnp.int32),
        session_table,
        subject_table,
        session_flag,
        subject_flag,
    )
